# baseline (device time: 162395 ns/iter reference)
import jax
import jax.numpy as jnp
from jax import lax
from jax.experimental import pallas as pl
from jax.experimental.pallas import tpu as pltpu

N_Z = 4
F_CHUNK = 512


def kernel(x, router, W1, W2):
    t_loc, d_model = x.shape
    e_loc, _, f_dim = W1.shape
    n_tok = N_Z * t_loc
    n_exp = N_Z * e_loc

    def body(x_hbm, router_hbm, w1_hbm, w2_hbm, out_ref,
             x_full, router_all, w1_buf, w2_buf, acc_ref, rs_send, rs_recv,
             ag_x_send, ag_x_recv, r_send_sems, r_recv_sems,
             rs_send_sems, rs_recv_sems, in_x_sem, in_r_sem, w1_sem, w2_sem):
        zi = lax.axis_index("z")
        xi = lax.axis_index("x")
        yi = lax.axis_index("y")
        left = (zi - 1) % N_Z
        right = (zi + 1) % N_Z

        cpx = pltpu.make_async_copy(
            x_hbm, x_full.at[pl.ds(zi * t_loc, t_loc)], in_x_sem)
        cpr = pltpu.make_async_copy(router_hbm, router_all.at[zi], in_r_sem)
        cp1 = pltpu.make_async_copy(w1_hbm.at[0], w1_buf.at[0], w1_sem.at[0])
        cp2 = pltpu.make_async_copy(w2_hbm.at[0], w2_buf.at[0], w2_sem.at[0])
        cpx.start()
        cpr.start()
        cp1.start()
        cp2.start()

        barrier = pltpu.get_barrier_semaphore()
        for nbr in (left, right):
            pl.semaphore_signal(
                barrier, inc=1,
                device_id=(xi, yi, nbr),
                device_id_type=pl.DeviceIdType.MESH,
            )
        pl.semaphore_wait(barrier, 2)
        cpx.wait()
        cpr.wait()

        pending_sends = []

        for k in range(N_Z - 1):
            peer = (zi + 1 + k) % N_Z
            rr = pltpu.make_async_remote_copy(
                src_ref=router_all.at[zi],
                dst_ref=router_all.at[zi],
                send_sem=r_send_sems.at[k],
                recv_sem=r_recv_sems.at[k],
                device_id=(xi, yi, peer),
                device_id_type=pl.DeviceIdType.MESH,
            )
            rr.start()
            pending_sends.append(rr)

        def ag_rdma(h):
            o = (zi - h) % N_Z
            return pltpu.make_async_remote_copy(
                src_ref=x_full.at[pl.ds(o * t_loc, t_loc)],
                dst_ref=x_full.at[pl.ds(o * t_loc, t_loc)],
                send_sem=ag_x_send.at[h],
                recv_sem=ag_x_recv.at[h],
                device_id=(xi, yi, right),
                device_id_type=pl.DeviceIdType.MESH,
            )

        ag0 = ag_rdma(0)
        ag0.start()
        pending_sends.append(ag0)

        for k in range(N_Z - 1):
            pltpu.make_async_remote_copy(
                src_ref=router_all.at[zi],
                dst_ref=router_all.at[zi],
                send_sem=r_send_sems.at[k],
                recv_sem=r_recv_sems.at[k],
                device_id=(xi, yi, left),
                device_id_type=pl.DeviceIdType.MESH,
            ).wait_recv()
        router_full = jnp.concatenate(
            [router_all[i] for i in range(N_Z)], axis=1)
        iota1 = lax.broadcasted_iota(jnp.int32, (t_loc, n_exp), 1)

        def expert_weight(rows_start, n_rows, iota, e_g):
            gates = jnp.dot(x_full[pl.ds(rows_start, n_rows)], router_full,
                            preferred_element_type=jnp.float32,
                            precision=lax.Precision.HIGHEST)
            m1 = jnp.max(gates, axis=1, keepdims=True)
            i1 = jnp.min(jnp.where(gates == m1, iota, n_exp),
                         axis=1, keepdims=True)
            g2 = jnp.where(iota == i1, jnp.finfo(jnp.float32).min, gates)
            m2 = jnp.max(g2, axis=1, keepdims=True)
            i2 = jnp.min(jnp.where(g2 == m2, iota, n_exp),
                         axis=1, keepdims=True)
            e2 = jnp.exp(m2 - m1)
            return (jnp.where(i1 == e_g, 1.0, 0.0)
                    + jnp.where(i2 == e_g, e2, 0.0)) / (1.0 + e2)

        def ffn(rows_start, n_rows, slot, wj, first):
            for fc in range(0, f_dim, F_CHUNK):
                h_act = jnp.maximum(
                    jnp.dot(x_full[pl.ds(rows_start, n_rows)],
                            w1_buf[slot, :, fc:fc + F_CHUNK],
                            preferred_element_type=jnp.float32), 0.0)
                contrib = jnp.dot(
                    h_act, w2_buf[slot, fc:fc + F_CHUNK, :],
                    preferred_element_type=jnp.float32) * wj
                if first and fc == 0:
                    acc_ref[pl.ds(rows_start, n_rows)] = contrib
                else:
                    acc_ref[pl.ds(rows_start, n_rows)] = (
                        acc_ref[pl.ds(rows_start, n_rows)] + contrib)

        def prefetch(j_next):
            nonlocal cp1, cp2
            nxt = j_next % 2
            cp1 = pltpu.make_async_copy(
                w1_hbm.at[j_next], w1_buf.at[nxt], w1_sem.at[nxt])
            cp2 = pltpu.make_async_copy(
                w2_hbm.at[j_next], w2_buf.at[nxt], w2_sem.at[nxt])
            cp1.start()
            cp2.start()

        cp1.wait()
        cp2.wait()
        prefetch(1)
        e0 = zi * e_loc
        for a in range(N_Z):
            o_a = (zi - a) % N_Z
            if a > 0:
                ag_rdma(a - 1).wait_recv()
            if 0 < a < N_Z - 1:
                nxt = ag_rdma(a)
                nxt.start()
                pending_sends.append(nxt)
            w0 = expert_weight(o_a * t_loc, t_loc, iota1, e0)
            ffn(o_a * t_loc, t_loc, 0, w0, first=True)

        iota_full = lax.broadcasted_iota(jnp.int32, (n_tok, n_exp), 1)
        for j in (1, 2):
            cp1.wait()
            cp2.wait()
            prefetch(j + 1)
            wj = expert_weight(0, n_tok, iota_full, zi * e_loc + j)
            ffn(0, n_tok, j % 2, wj, first=False)

        cp1.wait()
        cp2.wait()
        e3 = zi * e_loc + 3
        for s in range(N_Z - 1):
            c = (zi - 1 - s) % N_Z
            w3 = expert_weight(c * t_loc, t_loc, iota1, e3)
            ffn(c * t_loc, t_loc, 1, w3, first=False)
            data = acc_ref[pl.ds(c * t_loc, t_loc)]
            if s > 0:
                pltpu.make_async_remote_copy(
                    src_ref=rs_send.at[s - 1], dst_ref=rs_recv.at[s - 1],
                    send_sem=rs_send_sems.at[s - 1],
                    recv_sem=rs_recv_sems.at[s - 1],
                    device_id=(xi, yi, left),
                    device_id_type=pl.DeviceIdType.MESH,
                ).wait_recv()
                data = data + rs_recv[s - 1]
            rs_send[s] = data
            r = pltpu.make_async_remote_copy(
                src_ref=rs_send.at[s],
                dst_ref=rs_recv.at[s],
                send_sem=rs_send_sems.at[s],
                recv_sem=rs_recv_sems.at[s],
                device_id=(xi, yi, right),
                device_id_type=pl.DeviceIdType.MESH,
            )
            r.start()
            pending_sends.append(r)

        w3 = expert_weight(zi * t_loc, t_loc, iota1, e3)
        ffn(zi * t_loc, t_loc, 1, w3, first=False)
        pltpu.make_async_remote_copy(
            src_ref=rs_send.at[N_Z - 2], dst_ref=rs_recv.at[N_Z - 2],
            send_sem=rs_send_sems.at[N_Z - 2],
            recv_sem=rs_recv_sems.at[N_Z - 2],
            device_id=(xi, yi, left),
            device_id_type=pl.DeviceIdType.MESH,
        ).wait_recv()
        out_ref[...] = (
            acc_ref[pl.ds(zi * t_loc, t_loc)] + rs_recv[N_Z - 2])

        for snd in pending_sends:
            snd.wait_send()

    return pl.pallas_call(
        body,
        out_shape=jax.ShapeDtypeStruct((t_loc, d_model), jnp.float32),
        in_specs=[
            pl.BlockSpec(memory_space=pl.ANY),
            pl.BlockSpec(memory_space=pl.ANY),
            pl.BlockSpec(memory_space=pl.ANY),
            pl.BlockSpec(memory_space=pl.ANY),
        ],
        out_specs=pl.BlockSpec(memory_space=pltpu.VMEM),
        scratch_shapes=[
            pltpu.VMEM((n_tok, d_model), jnp.float32),
            pltpu.VMEM((N_Z, n_tok, e_loc), jnp.float32),
            pltpu.VMEM((2, d_model, f_dim), jnp.float32),
            pltpu.VMEM((2, f_dim, d_model), jnp.float32),
            pltpu.VMEM((n_tok, d_model), jnp.float32),
            pltpu.VMEM((N_Z - 1, t_loc, d_model), jnp.float32),
            pltpu.VMEM((N_Z - 1, t_loc, d_model), jnp.float32),
            pltpu.SemaphoreType.DMA((N_Z - 1,)),
            pltpu.SemaphoreType.DMA((N_Z - 1,)),
            pltpu.SemaphoreType.DMA((N_Z - 1,)),
            pltpu.SemaphoreType.DMA((N_Z - 1,)),
            pltpu.SemaphoreType.DMA((N_Z - 1,)),
            pltpu.SemaphoreType.DMA((N_Z - 1,)),
            pltpu.SemaphoreType.DMA,
            pltpu.SemaphoreType.DMA,
            pltpu.SemaphoreType.DMA((2,)),
            pltpu.SemaphoreType.DMA((2,)),
        ],
        compiler_params=pltpu.CompilerParams(
            collective_id=0, vmem_limit_bytes=63 * 1024 * 1024),
    )(x, router, W1, W2)


# device time: 144974 ns/iter; 1.1202x vs baseline; 1.1202x over previous
import jax
import jax.numpy as jnp
from jax import lax
from jax.experimental import pallas as pl
from jax.experimental.pallas import tpu as pltpu

N_Z = 4
CAP = 256


def kernel(x, router, W1, W2):
    t_loc, d_model = x.shape
    e_loc, _, f_dim = W1.shape
    n_tok = N_Z * t_loc
    n_exp = N_Z * e_loc

    def body(x_hbm, router_hbm, w1_hbm, w2_hbm, out_ref,
             x_full, router_all, w1_buf, w2_buf, acc_ref, rs_send, rs_recv,
             ag_x_send, ag_x_recv, ag_r_send, ag_r_recv,
             rs_send_sems, rs_recv_sems, in_x_sem, in_r_sem, w1_sem, w2_sem):
        zi = lax.axis_index("z")
        xi = lax.axis_index("x")
        yi = lax.axis_index("y")
        left = (zi - 1) % N_Z
        right = (zi + 1) % N_Z

        cpx = pltpu.make_async_copy(
            x_hbm, x_full.at[pl.ds(zi * t_loc, t_loc)], in_x_sem)
        cpr = pltpu.make_async_copy(router_hbm, router_all.at[zi], in_r_sem)
        cp1 = pltpu.make_async_copy(w1_hbm.at[0], w1_buf.at[0], w1_sem.at[0])
        cp2 = pltpu.make_async_copy(w2_hbm.at[0], w2_buf.at[0], w2_sem.at[0])
        cpx.start()
        cpr.start()
        cp1.start()
        cp2.start()

        barrier = pltpu.get_barrier_semaphore()
        for nbr in (left, right):
            pl.semaphore_signal(
                barrier, inc=1,
                device_id=(xi, yi, nbr),
                device_id_type=pl.DeviceIdType.MESH,
            )
        pl.semaphore_wait(barrier, 2)
        cpx.wait()
        cpr.wait()

        for h in range(N_Z - 1):
            o = (zi - h) % N_Z
            rx = pltpu.make_async_remote_copy(
                src_ref=x_full.at[pl.ds(o * t_loc, t_loc)],
                dst_ref=x_full.at[pl.ds(o * t_loc, t_loc)],
                send_sem=ag_x_send.at[h],
                recv_sem=ag_x_recv.at[h],
                device_id=(xi, yi, right),
                device_id_type=pl.DeviceIdType.MESH,
            )
            rr = pltpu.make_async_remote_copy(
                src_ref=router_all.at[o],
                dst_ref=router_all.at[o],
                send_sem=ag_r_send.at[h],
                recv_sem=ag_r_recv.at[h],
                device_id=(xi, yi, right),
                device_id_type=pl.DeviceIdType.MESH,
            )
            rx.start()
            rr.start()
            rx.wait()
            rr.wait()

        router_full = jnp.concatenate(
            [router_all[i] for i in range(N_Z)], axis=1)
        gates = jnp.dot(x_full[...], router_full,
                        preferred_element_type=jnp.float32,
                        precision=lax.Precision.HIGHEST)
        iota = lax.broadcasted_iota(jnp.int32, (n_tok, n_exp), 1)
        m1 = jnp.max(gates, axis=1, keepdims=True)
        i1 = jnp.min(jnp.where(gates == m1, iota, n_exp),
                     axis=1, keepdims=True)
        neg = jnp.finfo(jnp.float32).min
        g2 = jnp.where(iota == i1, neg, gates)
        m2 = jnp.max(g2, axis=1, keepdims=True)
        i2 = jnp.min(jnp.where(g2 == m2, iota, n_exp),
                     axis=1, keepdims=True)
        e2 = jnp.exp(m2 - m1)
        w_top1 = 1.0 / (1.0 + e2)
        w_top2 = e2 / (1.0 + e2)

        sel = []
        wgt = []
        for j in range(e_loc):
            e_g = zi * e_loc + j
            s1 = i1 == e_g
            s2 = i2 == e_g
            sel.append(jnp.where(s1 | s2, 1.0, 0.0))
            wgt.append(jnp.where(s1, w_top1, 0.0)
                       + jnp.where(s2, w_top2, 0.0))
        mask4 = jnp.concatenate(sel, axis=1)

        r_iota = lax.broadcasted_iota(jnp.int32, (n_tok, n_tok), 0)
        c_iota = lax.broadcasted_iota(jnp.int32, (n_tok, n_tok), 1)
        ltri = jnp.where(c_iota <= r_iota, 1.0, 0.0)
        ranks4 = jnp.dot(ltri, mask4,
                         preferred_element_type=jnp.float32)

        cap_iota = lax.broadcasted_iota(
            jnp.int32, (n_tok, CAP), 1).astype(jnp.float32)

        for j in range(e_loc):
            slot = j % 2
            cp1.wait()
            cp2.wait()
            if j + 1 < e_loc:
                nxt = (j + 1) % 2
                cp1 = pltpu.make_async_copy(
                    w1_hbm.at[j + 1], w1_buf.at[nxt], w1_sem.at[nxt])
                cp2 = pltpu.make_async_copy(
                    w2_hbm.at[j + 1], w2_buf.at[nxt], w2_sem.at[nxt])
                cp1.start()
                cp2.start()
            pos = ranks4[:, j:j + 1] - 1.0
            hit = (pos == cap_iota) & (sel[j] > 0)
            pt = jnp.where(hit, 1.0, 0.0)
            pt_w = jnp.where(hit, wgt[j], 0.0)
            g_rows = lax.dot_general(
                pt, x_full[...],
                dimension_numbers=(((0,), (0,)), ((), ())),
                preferred_element_type=jnp.float32)
            h_act = jnp.maximum(
                jnp.dot(g_rows, w1_buf[slot],
                        preferred_element_type=jnp.float32), 0.0)
            y = jnp.dot(h_act, w2_buf[slot],
                        preferred_element_type=jnp.float32)
            contrib = jnp.dot(pt_w, y,
                              preferred_element_type=jnp.float32)
            if j == 0:
                acc_ref[...] = contrib
            else:
                acc_ref[...] = acc_ref[...] + contrib

        for s in range(N_Z - 1):
            c = (zi - 1 - s) % N_Z
            data = acc_ref[pl.ds(c * t_loc, t_loc)]
            if s > 0:
                data = data + rs_recv[s - 1]
            rs_send[...] = data
            r = pltpu.make_async_remote_copy(
                src_ref=rs_send,
                dst_ref=rs_recv.at[s],
                send_sem=rs_send_sems.at[s],
                recv_sem=rs_recv_sems.at[s],
                device_id=(xi, yi, right),
                device_id_type=pl.DeviceIdType.MESH,
            )
            r.start()
            r.wait()

        out_ref[...] = (
            acc_ref[pl.ds(zi * t_loc, t_loc)] + rs_recv[N_Z - 2])

    return pl.pallas_call(
        body,
        out_shape=jax.ShapeDtypeStruct((t_loc, d_model), jnp.float32),
        in_specs=[
            pl.BlockSpec(memory_space=pl.ANY),
            pl.BlockSpec(memory_space=pl.ANY),
            pl.BlockSpec(memory_space=pl.ANY),
            pl.BlockSpec(memory_space=pl.ANY),
        ],
        out_specs=pl.BlockSpec(memory_space=pltpu.VMEM),
        scratch_shapes=[
            pltpu.VMEM((n_tok, d_model), jnp.float32),
            pltpu.VMEM((N_Z, n_tok, e_loc), jnp.float32),
            pltpu.VMEM((2, d_model, f_dim), jnp.float32),
            pltpu.VMEM((2, f_dim, d_model), jnp.float32),
            pltpu.VMEM((n_tok, d_model), jnp.float32),
            pltpu.VMEM((t_loc, d_model), jnp.float32),
            pltpu.VMEM((N_Z - 1, t_loc, d_model), jnp.float32),
            pltpu.SemaphoreType.DMA((N_Z - 1,)),
            pltpu.SemaphoreType.DMA((N_Z - 1,)),
            pltpu.SemaphoreType.DMA((N_Z - 1,)),
            pltpu.SemaphoreType.DMA((N_Z - 1,)),
            pltpu.SemaphoreType.DMA((N_Z - 1,)),
            pltpu.SemaphoreType.DMA((N_Z - 1,)),
            pltpu.SemaphoreType.DMA,
            pltpu.SemaphoreType.DMA,
            pltpu.SemaphoreType.DMA((2,)),
            pltpu.SemaphoreType.DMA((2,)),
        ],
        compiler_params=pltpu.CompilerParams(
            collective_id=0, vmem_limit_bytes=63 * 1024 * 1024),
    )(x, router, W1, W2)
